# Initial kernel scaffold; baseline (speedup 1.0000x reference)
import jax
import jax.numpy as jnp
from jax.experimental import pallas as pl


def kernel(x, edge_index, batch_index, W1, b1, W2, b2, W3, b3):
    raise NotImplementedError("write your pallas kernel here")


# speedup vs baseline: 4.4823x; 4.4823x over previous
import functools

import jax
import jax.numpy as jnp
from jax import lax
from jax.experimental import pallas as pl
from jax.experimental.pallas import tpu as pltpu
from jax.experimental.pallas import tpu_sc as plsc

N = 10000
E = 160000
F_IN = 256
H = 512
F_OUT = 256
G = 64

NPAD = 10240
NC = 2
NS = 16
EPAD = NS * NPAD
EPS = EPAD // NS
CW = 128
NCH = EPS // CW
BM = 1024
NI = NPAD // BM
RPT = NPAD // NS

_f32 = jnp.float32
_i32 = jnp.int32


def _sc_mesh():
    return plsc.VectorSubcoreMesh(core_axis_name="c", subcore_axis_name="s")


def _deg_body(dst_hbm, ones_hbm, out_hbm, dst_v, vbuf, acc):
    c = lax.axis_index("c")
    s = lax.axis_index("s")
    pltpu.sync_copy(dst_hbm.at[c, s], dst_v)
    for t in range(RPT // CW):
        r0 = s * RPT + t * CW
        pltpu.sync_copy(ones_hbm.at[pl.ds(CW, CW)], vbuf)
        pltpu.sync_copy(vbuf, acc.at[pl.ds(r0, CW)])
    plsc.subcore_barrier()

    pltpu.sync_copy(ones_hbm.at[pl.ds(0, CW)], vbuf)
    nch = (EPAD // (NC * NS)) // CW

    def chunk(j, carry):
        pltpu.sync_copy(vbuf, acc.at[dst_v.at[j]], add=True)
        return carry

    lax.fori_loop(0, nch, chunk, 0)
    plsc.subcore_barrier()
    for t in range(RPT // CW):
        r0 = s * RPT + t * CW
        pltpu.sync_copy(acc.at[pl.ds(r0, CW)], vbuf)
        pltpu.sync_copy(vbuf, out_hbm.at[c, pl.ds(r0, CW)])


def _deg_call(dst_p):
    dst_r = dst_p.reshape(NC, NS, (EPAD // (NC * NS)) // CW, CW)
    ones = jnp.concatenate([jnp.ones((CW, 128), _f32),
                            jnp.zeros((CW, 128), _f32)])
    f = pl.kernel(
        _deg_body,
        out_type=jax.ShapeDtypeStruct((NC, NPAD, 128), _f32),
        mesh=_sc_mesh(),
        scratch_types=[
            pltpu.VMEM(((EPAD // (NC * NS)) // CW, CW), _i32),
            pltpu.VMEM((CW, 128), _f32),
            pltpu.VMEM_SHARED((NPAD, 128), _f32),
        ],
    )
    return f(dst_r, ones)


def _prop_body(nb, y_hbm, src_hbm, dst_hbm, out_hbm,
               src_v, srcs_v, dst_v, gbuf, acc):
    c = lax.axis_index("c")
    s = lax.axis_index("s")
    pltpu.sync_copy(src_hbm.at[s], src_v)
    pltpu.sync_copy(dst_hbm.at[s], dst_v)

    for bi in range(nb // NC):
        bb = NC * bi + c
        base = bb * NPAD

        def shift(i, carry):
            sl = pl.ds(i * 16, 16)
            srcs_v[sl] = src_v[sl] + base
            return carry

        lax.fori_loop(0, EPS // 16, shift, 0)

        if bi > 0:
            plsc.subcore_barrier()
        for t in range(RPT // CW):
            r0 = s * RPT + t * CW
            pltpu.sync_copy(y_hbm.at[pl.ds(base + r0, CW)], gbuf)
            pltpu.sync_copy(gbuf, acc.at[pl.ds(r0, CW)])
        plsc.subcore_barrier()

        def chunk(j, carry):
            pltpu.sync_copy(y_hbm.at[srcs_v.at[pl.ds(j * CW, CW)]], gbuf)
            pltpu.sync_copy(gbuf, acc.at[dst_v.at[j]], add=True)
            return carry

        lax.fori_loop(0, NCH, chunk, 0)
        plsc.subcore_barrier()
        for t in range(RPT // CW):
            r0 = s * RPT + t * CW
            pltpu.sync_copy(acc.at[pl.ds(r0, CW)], gbuf)
            pltpu.sync_copy(gbuf, out_hbm.at[pl.ds(base + r0, CW)])


def _prop_call(y_flat, src_r, dst_r, nb):
    f = pl.kernel(
        functools.partial(_prop_body, nb),
        out_type=jax.ShapeDtypeStruct((nb * NPAD, 128), _f32),
        mesh=_sc_mesh(),
        scratch_types=[
            pltpu.VMEM((EPS,), _i32),
            pltpu.VMEM((EPS,), _i32),
            pltpu.VMEM((NCH, CW), _i32),
            pltpu.VMEM((CW, 128), _f32),
            pltpu.VMEM_SHARED((NPAD, 128), _f32),
        ],
    )
    return f(y_flat, src_r, dst_r)


def _mm_first_body(x_ref, w_ref, deg_ref, y_ref, dinv_ref, *, nk):
    k = pl.program_id(2)
    part = jnp.dot(x_ref[...], w_ref[...], preferred_element_type=_f32,
                   precision=lax.Precision.HIGHEST)

    @pl.when(k == 0)
    def _():
        y_ref[0] = part

    @pl.when(k > 0)
    def _():
        y_ref[0] = y_ref[0] + part

    @pl.when(k == nk - 1)
    def _():
        i = pl.program_id(0)
        deg = jnp.sum(deg_ref[...], axis=(0, 2)) * (1.0 / 128.0) + 1.0
        row = lax.broadcasted_iota(_i32, (BM, 1), 0) + i * BM
        dv = jnp.where(row < N, lax.rsqrt(deg)[:, None], 0.0)
        y_ref[0] = y_ref[0] * dv
        dinv_ref[...] = dv


def _mm_first(x_p, W1, deg2):
    nj, nk = H // 128, F_IN // 128
    return pl.pallas_call(
        functools.partial(_mm_first_body, nk=nk),
        grid=(NI, nj, nk),
        in_specs=[
            pl.BlockSpec((BM, 128), lambda i, j, k: (i, k)),
            pl.BlockSpec((128, 128), lambda i, j, k: (k, j)),
            pl.BlockSpec((NC, BM, 128), lambda i, j, k: (0, i, 0)),
        ],
        out_specs=[
            pl.BlockSpec((1, BM, 128), lambda i, j, k: (j, i, 0)),
            pl.BlockSpec((BM, 1), lambda i, j, k: (i, 0)),
        ],
        out_shape=[
            jax.ShapeDtypeStruct((nj, NPAD, 128), _f32),
            jax.ShapeDtypeStruct((NPAD, 1), _f32),
        ],
    )(x_p, W1, deg2)


def _mm_mid_body(s_ref, w_ref, b_ref, dinv_ref, y_ref, *, nk):
    k = pl.program_id(2)
    h = jnp.maximum(s_ref[0] * dinv_ref[...] + b_ref[...], 0.0)
    part = jnp.dot(h, w_ref[...], preferred_element_type=_f32,
                   precision=lax.Precision.HIGHEST)

    @pl.when(k == 0)
    def _():
        y_ref[0] = part

    @pl.when(k > 0)
    def _():
        y_ref[0] = y_ref[0] + part

    @pl.when(k == nk - 1)
    def _():
        y_ref[0] = y_ref[0] * dinv_ref[...]


def _mm_mid(s_in, W, b_prev, dinv):
    nk = s_in.shape[0]
    nj = W.shape[1] // 128
    return pl.pallas_call(
        functools.partial(_mm_mid_body, nk=nk),
        grid=(NI, nj, nk),
        in_specs=[
            pl.BlockSpec((1, BM, 128), lambda i, j, k: (k, i, 0)),
            pl.BlockSpec((128, 128), lambda i, j, k: (k, j)),
            pl.BlockSpec((1, 128), lambda i, j, k: (0, k)),
            pl.BlockSpec((BM, 1), lambda i, j, k: (i, 0)),
        ],
        out_specs=pl.BlockSpec((1, BM, 128), lambda i, j, k: (j, i, 0)),
        out_shape=jax.ShapeDtypeStruct((nj, NPAD, 128), _f32),
    )(s_in, W, b_prev.reshape(1, -1), dinv)


def _pool_body(s_ref, dinv_ref, b3_ref, bidx_ref, out_ref, acc_ref, cnt_ref):
    i = pl.program_id(0)

    @pl.when(i == 0)
    def _():
        acc_ref[...] = jnp.zeros_like(acc_ref)
        cnt_ref[...] = jnp.zeros_like(cnt_ref)

    oh = (bidx_ref[...] == lax.broadcasted_iota(_i32, (BM, G), 1)).astype(_f32)
    dn = (((0,), (0,)), ((), ()))
    v0 = s_ref[0] * dinv_ref[...]
    v1 = s_ref[1] * dinv_ref[...]
    acc_ref[:, :128] = acc_ref[:, :128] + lax.dot_general(
        oh, v0, dn, preferred_element_type=_f32, precision=lax.Precision.HIGHEST)
    acc_ref[:, 128:] = acc_ref[:, 128:] + lax.dot_general(
        oh, v1, dn, preferred_element_type=_f32, precision=lax.Precision.HIGHEST)
    cnt_ref[...] = cnt_ref[...] + lax.dot_general(
        oh, jnp.ones((BM, 128), _f32), dn, preferred_element_type=_f32,
        precision=lax.Precision.HIGHEST)

    @pl.when(i == NI - 1)
    def _():
        cnt = cnt_ref[:, :1]
        out_ref[...] = (acc_ref[...] + cnt * b3_ref[...]) / jnp.maximum(cnt, 1.0)


def _pool(s3, dinv, b3, batch_p):
    return pl.pallas_call(
        _pool_body,
        grid=(NI,),
        in_specs=[
            pl.BlockSpec((2, BM, 128), lambda i: (0, i, 0)),
            pl.BlockSpec((BM, 1), lambda i: (i, 0)),
            pl.BlockSpec((1, F_OUT), lambda i: (0, 0)),
            pl.BlockSpec((BM, 1), lambda i: (i, 0)),
        ],
        out_specs=pl.BlockSpec((G, F_OUT), lambda i: (0, 0)),
        out_shape=jax.ShapeDtypeStruct((G, F_OUT), _f32),
        scratch_shapes=[
            pltpu.VMEM((G, F_OUT), _f32),
            pltpu.VMEM((G, 128), _f32),
        ],
    )(s3, dinv, b3.reshape(1, F_OUT), batch_p)


def kernel(x, edge_index, batch_index, W1, b1, W2, b2, W3, b3):
    src = edge_index[0]
    dst = edge_index[1]
    pad = jnp.full((EPAD - E,), NPAD - 1, _i32)
    src_p = jnp.concatenate([src, pad])
    dst_p = jnp.concatenate([dst, pad])
    src_r = src_p.reshape(NS, EPS)
    dst_r = dst_p.reshape(NS, NCH, CW)

    x_p = jnp.pad(x, ((0, NPAD - N), (0, 0)))
    batch_p = jnp.pad(batch_index, (0, NPAD - N), constant_values=G)
    batch_p = batch_p.reshape(NPAD, 1)

    deg2 = _deg_call(dst_p)
    y1, dinv = _mm_first(x_p, W1, deg2)
    s1 = _prop_call(y1.reshape(-1, 128), src_r, dst_r, nb=4)
    y2 = _mm_mid(s1.reshape(4, NPAD, 128), W2, b1, dinv)
    s2 = _prop_call(y2.reshape(-1, 128), src_r, dst_r, nb=4)
    y3 = _mm_mid(s2.reshape(4, NPAD, 128), W3, b2, dinv)
    s3 = _prop_call(y3.reshape(-1, 128), src_r, dst_r, nb=2)
    return _pool(s3.reshape(2, NPAD, 128), dinv, b3, batch_p)


# speedup vs baseline: 4.8378x; 1.0793x over previous
import functools

import jax
import jax.numpy as jnp
from jax import lax
from jax.experimental import pallas as pl
from jax.experimental.pallas import tpu as pltpu
from jax.experimental.pallas import tpu_sc as plsc

N = 10000
E = 160000
F_IN = 256
H = 512
F_OUT = 256
G = 64

NPAD = 10240
NC = 2
NS = 16
EPAD = NS * NPAD
EPS = EPAD // NS
CW = 128
NCH = EPS // CW
BM = 1024
NI = NPAD // BM
RPT = NPAD // NS

_f32 = jnp.float32
_i32 = jnp.int32


def _sc_mesh():
    return plsc.VectorSubcoreMesh(core_axis_name="c", subcore_axis_name="s")


def _deg_body(dst_hbm, ones_hbm, out_hbm, dst_v, vbuf, acc):
    c = lax.axis_index("c")
    s = lax.axis_index("s")
    pltpu.sync_copy(dst_hbm.at[c, s], dst_v)
    for t in range(RPT // CW):
        r0 = s * RPT + t * CW
        pltpu.sync_copy(ones_hbm.at[pl.ds(CW, CW)], vbuf)
        pltpu.sync_copy(vbuf, acc.at[pl.ds(r0, CW)])
    plsc.subcore_barrier()

    pltpu.sync_copy(ones_hbm.at[pl.ds(0, CW)], vbuf)
    nch = (EPAD // (NC * NS)) // CW

    def chunk(j, carry):
        pltpu.sync_copy(vbuf, acc.at[dst_v.at[j]], add=True)
        return carry

    lax.fori_loop(0, nch, chunk, 0)
    plsc.subcore_barrier()
    for t in range(RPT // CW):
        r0 = s * RPT + t * CW
        pltpu.sync_copy(acc.at[pl.ds(r0, CW)], vbuf)
        pltpu.sync_copy(vbuf, out_hbm.at[c, pl.ds(r0, CW)])


def _deg_call(dst_p):
    dst_r = dst_p.reshape(NC, NS, (EPAD // (NC * NS)) // CW, CW)
    ones = jnp.concatenate([jnp.ones((CW, 128), _f32),
                            jnp.zeros((CW, 128), _f32)])
    f = pl.kernel(
        _deg_body,
        out_type=jax.ShapeDtypeStruct((NC, NPAD, 128), _f32),
        mesh=_sc_mesh(),
        scratch_types=[
            pltpu.VMEM(((EPAD // (NC * NS)) // CW, CW), _i32),
            pltpu.VMEM((CW, 128), _f32),
            pltpu.VMEM_SHARED((NPAD, 128), _f32),
        ],
    )
    return f(dst_r, ones)


NHALF = 2
EPH = EPS // NHALF
NCHH = EPH // CW


def _prop_body(nb, y_hbm, src_hbm, dst_hbm, out_hbm,
               srcs_v, dst_v, gbuf, gbuf2, acc, sem_a, sem_b):
    c = lax.axis_index("c")
    s = lax.axis_index("s")

    for bi in range(nb // NC):
        bb = NC * bi + c
        base = bb * NPAD

        if bi > 0:
            plsc.subcore_barrier()
        for t in range(RPT // CW):
            r0 = s * RPT + t * CW
            pltpu.sync_copy(y_hbm.at[pl.ds(base + r0, CW)], gbuf)
            pltpu.sync_copy(gbuf, acc.at[pl.ds(r0, CW)])
        plsc.subcore_barrier()

        for h in range(NHALF):
            pltpu.sync_copy(src_hbm.at[s, h], srcs_v)
            pltpu.sync_copy(dst_hbm.at[s, h], dst_v)

            def shift(i, carry):
                sl = pl.ds(i * 16, 16)
                srcs_v[sl] = srcs_v[sl] + base
                return carry

            lax.fori_loop(0, EPH // 16, shift, 0)

            pltpu.async_copy(y_hbm.at[srcs_v.at[pl.ds(0, CW)]], gbuf, sem_a)

            def pair(p, carry):
                j0 = 2 * p
                pltpu.make_async_copy(
                    y_hbm.at[srcs_v.at[pl.ds(j0 * CW, CW)]], gbuf, sem_a).wait()
                pltpu.async_copy(
                    y_hbm.at[srcs_v.at[pl.ds((j0 + 1) * CW, CW)]], gbuf2, sem_b)
                pltpu.sync_copy(gbuf, acc.at[dst_v.at[j0]], add=True)
                pltpu.make_async_copy(
                    y_hbm.at[srcs_v.at[pl.ds((j0 + 1) * CW, CW)]], gbuf2,
                    sem_b).wait()
                nxt = jnp.minimum(j0 + 2, NCHH - 1) * CW
                pltpu.async_copy(y_hbm.at[srcs_v.at[pl.ds(nxt, CW)]], gbuf,
                                 sem_a)
                pltpu.sync_copy(gbuf2, acc.at[dst_v.at[j0 + 1]], add=True)
                return carry

            lax.fori_loop(0, NCHH // 2, pair, 0)
            pltpu.make_async_copy(
                y_hbm.at[srcs_v.at[pl.ds((NCHH - 1) * CW, CW)]], gbuf,
                sem_a).wait()
        plsc.subcore_barrier()
        for t in range(RPT // CW):
            r0 = s * RPT + t * CW
            pltpu.sync_copy(acc.at[pl.ds(r0, CW)], gbuf)
            pltpu.sync_copy(gbuf, out_hbm.at[pl.ds(base + r0, CW)])


def _prop_call(y_flat, src_r, dst_r, nb):
    f = pl.kernel(
        functools.partial(_prop_body, nb),
        out_type=jax.ShapeDtypeStruct((nb * NPAD, 128), _f32),
        mesh=_sc_mesh(),
        scratch_types=[
            pltpu.VMEM((EPH,), _i32),
            pltpu.VMEM((NCHH, CW), _i32),
            pltpu.VMEM((CW, 128), _f32),
            pltpu.VMEM((CW, 128), _f32),
            pltpu.VMEM_SHARED((NPAD, 128), _f32),
            pltpu.SemaphoreType.DMA,
            pltpu.SemaphoreType.DMA,
        ],
    )
    return f(y_flat, src_r, dst_r)


def _mm_first_body(x_ref, w_ref, deg_ref, y_ref, dinv_ref, *, nk):
    k = pl.program_id(2)
    part = jnp.dot(x_ref[...], w_ref[...], preferred_element_type=_f32,
                   precision=lax.Precision.HIGHEST)

    @pl.when(k == 0)
    def _():
        y_ref[0] = part

    @pl.when(k > 0)
    def _():
        y_ref[0] = y_ref[0] + part

    @pl.when(k == nk - 1)
    def _():
        i = pl.program_id(0)
        deg = jnp.sum(deg_ref[...], axis=(0, 2)) * (1.0 / 128.0) + 1.0
        row = lax.broadcasted_iota(_i32, (BM, 1), 0) + i * BM
        dv = jnp.where(row < N, lax.rsqrt(deg)[:, None], 0.0)
        y_ref[0] = y_ref[0] * dv
        dinv_ref[...] = dv


def _mm_first(x_p, W1, deg2):
    nj, nk = H // 128, F_IN // 128
    return pl.pallas_call(
        functools.partial(_mm_first_body, nk=nk),
        grid=(NI, nj, nk),
        in_specs=[
            pl.BlockSpec((BM, 128), lambda i, j, k: (i, k)),
            pl.BlockSpec((128, 128), lambda i, j, k: (k, j)),
            pl.BlockSpec((NC, BM, 128), lambda i, j, k: (0, i, 0)),
        ],
        out_specs=[
            pl.BlockSpec((1, BM, 128), lambda i, j, k: (j, i, 0)),
            pl.BlockSpec((BM, 1), lambda i, j, k: (i, 0)),
        ],
        out_shape=[
            jax.ShapeDtypeStruct((nj, NPAD, 128), _f32),
            jax.ShapeDtypeStruct((NPAD, 1), _f32),
        ],
    )(x_p, W1, deg2)


def _mm_mid_body(s_ref, w_ref, b_ref, dinv_ref, y_ref, *, nk):
    k = pl.program_id(2)
    h = jnp.maximum(s_ref[0] * dinv_ref[...] + b_ref[...], 0.0)
    part = jnp.dot(h, w_ref[...], preferred_element_type=_f32,
                   precision=lax.Precision.HIGHEST)

    @pl.when(k == 0)
    def _():
        y_ref[0] = part

    @pl.when(k > 0)
    def _():
        y_ref[0] = y_ref[0] + part

    @pl.when(k == nk - 1)
    def _():
        y_ref[0] = y_ref[0] * dinv_ref[...]


def _mm_mid(s_in, W, b_prev, dinv):
    nk = s_in.shape[0]
    nj = W.shape[1] // 128
    return pl.pallas_call(
        functools.partial(_mm_mid_body, nk=nk),
        grid=(NI, nj, nk),
        in_specs=[
            pl.BlockSpec((1, BM, 128), lambda i, j, k: (k, i, 0)),
            pl.BlockSpec((128, 128), lambda i, j, k: (k, j)),
            pl.BlockSpec((1, 128), lambda i, j, k: (0, k)),
            pl.BlockSpec((BM, 1), lambda i, j, k: (i, 0)),
        ],
        out_specs=pl.BlockSpec((1, BM, 128), lambda i, j, k: (j, i, 0)),
        out_shape=jax.ShapeDtypeStruct((nj, NPAD, 128), _f32),
    )(s_in, W, b_prev.reshape(1, -1), dinv)


def _pool_body(s_ref, dinv_ref, b3_ref, bidx_ref, out_ref, acc_ref, cnt_ref):
    i = pl.program_id(0)

    @pl.when(i == 0)
    def _():
        acc_ref[...] = jnp.zeros_like(acc_ref)
        cnt_ref[...] = jnp.zeros_like(cnt_ref)

    oh = (bidx_ref[...] == lax.broadcasted_iota(_i32, (BM, G), 1)).astype(_f32)
    dn = (((0,), (0,)), ((), ()))
    v0 = s_ref[0] * dinv_ref[...]
    v1 = s_ref[1] * dinv_ref[...]
    acc_ref[:, :128] = acc_ref[:, :128] + lax.dot_general(
        oh, v0, dn, preferred_element_type=_f32, precision=lax.Precision.HIGHEST)
    acc_ref[:, 128:] = acc_ref[:, 128:] + lax.dot_general(
        oh, v1, dn, preferred_element_type=_f32, precision=lax.Precision.HIGHEST)
    cnt_ref[...] = cnt_ref[...] + lax.dot_general(
        oh, jnp.ones((BM, 128), _f32), dn, preferred_element_type=_f32,
        precision=lax.Precision.HIGHEST)

    @pl.when(i == NI - 1)
    def _():
        cnt = cnt_ref[:, :1]
        out_ref[...] = (acc_ref[...] + cnt * b3_ref[...]) / jnp.maximum(cnt, 1.0)


def _pool(s3, dinv, b3, batch_p):
    return pl.pallas_call(
        _pool_body,
        grid=(NI,),
        in_specs=[
            pl.BlockSpec((2, BM, 128), lambda i: (0, i, 0)),
            pl.BlockSpec((BM, 1), lambda i: (i, 0)),
            pl.BlockSpec((1, F_OUT), lambda i: (0, 0)),
            pl.BlockSpec((BM, 1), lambda i: (i, 0)),
        ],
        out_specs=pl.BlockSpec((G, F_OUT), lambda i: (0, 0)),
        out_shape=jax.ShapeDtypeStruct((G, F_OUT), _f32),
        scratch_shapes=[
            pltpu.VMEM((G, F_OUT), _f32),
            pltpu.VMEM((G, 128), _f32),
        ],
    )(s3, dinv, b3.reshape(1, F_OUT), batch_p)


def kernel(x, edge_index, batch_index, W1, b1, W2, b2, W3, b3):
    src = edge_index[0]
    dst = edge_index[1]
    pad = jnp.full((EPAD - E,), NPAD - 1, _i32)
    src_p = jnp.concatenate([src, pad])
    dst_p = jnp.concatenate([dst, pad])
    src_r = src_p.reshape(NS, NHALF, EPH)
    dst_r = dst_p.reshape(NS, NHALF, NCHH, CW)

    x_p = jnp.pad(x, ((0, NPAD - N), (0, 0)))
    batch_p = jnp.pad(batch_index, (0, NPAD - N), constant_values=G)
    batch_p = batch_p.reshape(NPAD, 1)

    deg2 = _deg_call(dst_p)
    y1, dinv = _mm_first(x_p, W1, deg2)
    s1 = _prop_call(y1.reshape(-1, 128), src_r, dst_r, nb=4)
    y2 = _mm_mid(s1.reshape(4, NPAD, 128), W2, b1, dinv)
    s2 = _prop_call(y2.reshape(-1, 128), src_r, dst_r, nb=4)
    y3 = _mm_mid(s2.reshape(4, NPAD, 128), W3, b2, dinv)
    s3 = _prop_call(y3.reshape(-1, 128), src_r, dst_r, nb=2)
    return _pool(s3.reshape(2, NPAD, 128), dinv, b3, batch_p)


# speedup vs baseline: 4.9607x; 1.0254x over previous
import functools

import jax
import jax.numpy as jnp
from jax import lax
from jax.experimental import pallas as pl
from jax.experimental.pallas import tpu as pltpu
from jax.experimental.pallas import tpu_sc as plsc

N = 10000
E = 160000
F_IN = 256
H = 512
F_OUT = 256
G = 64

NPAD = 10240
NC = 2
NS = 16
EPAD = NS * NPAD
EPS = EPAD // NS
CW = 128
NCH = EPS // CW
BM = 1024
NI = NPAD // BM
RPT = NPAD // NS

_f32 = jnp.float32
_i32 = jnp.int32


def _sc_mesh():
    return plsc.VectorSubcoreMesh(core_axis_name="c", subcore_axis_name="s")


def _deg_body(dst_hbm, ones_hbm, out_hbm, dst_v, vbuf, acc):
    c = lax.axis_index("c")
    s = lax.axis_index("s")
    pltpu.sync_copy(dst_hbm.at[c, s], dst_v)
    for t in range(RPT // CW):
        r0 = s * RPT + t * CW
        pltpu.sync_copy(ones_hbm.at[pl.ds(CW, CW)], vbuf)
        pltpu.sync_copy(vbuf, acc.at[pl.ds(r0, CW)])
    plsc.subcore_barrier()

    pltpu.sync_copy(ones_hbm.at[pl.ds(0, CW)], vbuf)
    nch = (EPAD // (NC * NS)) // CW

    def chunk(j, carry):
        pltpu.sync_copy(vbuf, acc.at[dst_v.at[j]], add=True)
        return carry

    lax.fori_loop(0, nch, chunk, 0)
    plsc.subcore_barrier()
    for t in range(RPT // CW):
        r0 = s * RPT + t * CW
        pltpu.sync_copy(acc.at[pl.ds(r0, CW)], vbuf)
        pltpu.sync_copy(vbuf, out_hbm.at[c, pl.ds(r0, CW)])


def _deg_call(dst_p):
    dst_r = dst_p.reshape(NC, NS, (EPAD // (NC * NS)) // CW, CW)
    ones = jnp.concatenate([jnp.ones((CW, 128), _f32),
                            jnp.zeros((CW, 128), _f32)])
    f = pl.kernel(
        _deg_body,
        out_type=jax.ShapeDtypeStruct((NC, NPAD, 128), _f32),
        mesh=_sc_mesh(),
        scratch_types=[
            pltpu.VMEM(((EPAD // (NC * NS)) // CW, CW), _i32),
            pltpu.VMEM((CW, 128), _f32),
            pltpu.VMEM_SHARED((NPAD, 128), _f32),
        ],
    )
    return f(dst_r, ones)


NHALF = 2
EPH = EPS // NHALF
NCHH = EPH // CW


def _prop_body(nb, y_hbm, src_hbm, dst_hbm, out_hbm,
               srcs_v, dst_v, gbuf, gbuf2, acc, sem_a, sem_b, sem_sa, sem_sb):
    c = lax.axis_index("c")
    s = lax.axis_index("s")

    for bi in range(nb // NC):
        bb = NC * bi + c
        base = bb * NPAD

        if bi > 0:
            plsc.subcore_barrier()
        for t in range(RPT // CW):
            r0 = s * RPT + t * CW
            pltpu.sync_copy(y_hbm.at[pl.ds(base + r0, CW)], gbuf)
            pltpu.sync_copy(gbuf, acc.at[pl.ds(r0, CW)])
        plsc.subcore_barrier()

        for h in range(NHALF):
            pltpu.sync_copy(src_hbm.at[s, h], srcs_v)
            pltpu.sync_copy(dst_hbm.at[s, h], dst_v)

            def shift(i, carry):
                sl = pl.ds(i * 16, 16)
                srcs_v[sl] = srcs_v[sl] + base
                return carry

            lax.fori_loop(0, EPH // 16, shift, 0)

            pltpu.async_copy(y_hbm.at[srcs_v.at[pl.ds(0, CW)]], gbuf, sem_a)

            def pair(p, carry):
                j0 = 2 * p
                pltpu.make_async_copy(
                    y_hbm.at[srcs_v.at[pl.ds(j0 * CW, CW)]], gbuf, sem_a).wait()

                @pl.when(p > 0)
                def _():
                    pltpu.make_async_copy(
                        gbuf2, acc.at[dst_v.at[j0 - 1]], sem_sb).wait()

                pltpu.async_copy(
                    y_hbm.at[srcs_v.at[pl.ds((j0 + 1) * CW, CW)]], gbuf2, sem_b)
                pltpu.async_copy(gbuf, acc.at[dst_v.at[j0]], sem_sa, add=True)
                pltpu.make_async_copy(
                    y_hbm.at[srcs_v.at[pl.ds((j0 + 1) * CW, CW)]], gbuf2,
                    sem_b).wait()
                pltpu.make_async_copy(
                    gbuf, acc.at[dst_v.at[j0]], sem_sa).wait()
                nxt = jnp.minimum(j0 + 2, NCHH - 1) * CW
                pltpu.async_copy(y_hbm.at[srcs_v.at[pl.ds(nxt, CW)]], gbuf,
                                 sem_a)
                pltpu.async_copy(gbuf2, acc.at[dst_v.at[j0 + 1]], sem_sb,
                                 add=True)
                return carry

            lax.fori_loop(0, NCHH // 2, pair, 0)
            pltpu.make_async_copy(
                y_hbm.at[srcs_v.at[pl.ds((NCHH - 1) * CW, CW)]], gbuf,
                sem_a).wait()
            pltpu.make_async_copy(
                gbuf2, acc.at[dst_v.at[NCHH - 1]], sem_sb).wait()
        plsc.subcore_barrier()
        for t in range(RPT // CW):
            r0 = s * RPT + t * CW
            pltpu.sync_copy(acc.at[pl.ds(r0, CW)], gbuf)
            pltpu.sync_copy(gbuf, out_hbm.at[pl.ds(base + r0, CW)])


def _prop_call(y_flat, src_r, dst_r, nb):
    f = pl.kernel(
        functools.partial(_prop_body, nb),
        out_type=jax.ShapeDtypeStruct((nb * NPAD, 128), _f32),
        mesh=_sc_mesh(),
        scratch_types=[
            pltpu.VMEM((EPH,), _i32),
            pltpu.VMEM((NCHH, CW), _i32),
            pltpu.VMEM((CW, 128), _f32),
            pltpu.VMEM((CW, 128), _f32),
            pltpu.VMEM_SHARED((NPAD, 128), _f32),
            pltpu.SemaphoreType.DMA,
            pltpu.SemaphoreType.DMA,
            pltpu.SemaphoreType.DMA,
            pltpu.SemaphoreType.DMA,
        ],
    )
    return f(y_flat, src_r, dst_r)


def _mm_first_body(x_ref, w_ref, deg_ref, y_ref, dinv_ref, *, nk):
    k = pl.program_id(2)
    part = jnp.dot(x_ref[...], w_ref[...], preferred_element_type=_f32,
                   precision=lax.Precision.DEFAULT)

    @pl.when(k == 0)
    def _():
        y_ref[0] = part

    @pl.when(k > 0)
    def _():
        y_ref[0] = y_ref[0] + part

    @pl.when(k == nk - 1)
    def _():
        i = pl.program_id(0)
        deg = jnp.sum(deg_ref[...], axis=(0, 2)) * (1.0 / 128.0) + 1.0
        row = lax.broadcasted_iota(_i32, (BM, 1), 0) + i * BM
        dv = jnp.where(row < N, lax.rsqrt(deg)[:, None], 0.0)
        y_ref[0] = y_ref[0] * dv
        dinv_ref[...] = dv


def _mm_first(x_p, W1, deg2):
    nj, nk = H // 128, F_IN // 128
    return pl.pallas_call(
        functools.partial(_mm_first_body, nk=nk),
        grid=(NI, nj, nk),
        in_specs=[
            pl.BlockSpec((BM, 128), lambda i, j, k: (i, k)),
            pl.BlockSpec((128, 128), lambda i, j, k: (k, j)),
            pl.BlockSpec((NC, BM, 128), lambda i, j, k: (0, i, 0)),
        ],
        out_specs=[
            pl.BlockSpec((1, BM, 128), lambda i, j, k: (j, i, 0)),
            pl.BlockSpec((BM, 1), lambda i, j, k: (i, 0)),
        ],
        out_shape=[
            jax.ShapeDtypeStruct((nj, NPAD, 128), _f32),
            jax.ShapeDtypeStruct((NPAD, 1), _f32),
        ],
    )(x_p, W1, deg2)


def _mm_mid_body(s_ref, w_ref, b_ref, dinv_ref, y_ref, *, nk):
    k = pl.program_id(2)
    h = jnp.maximum(s_ref[0] * dinv_ref[...] + b_ref[...], 0.0)
    part = jnp.dot(h, w_ref[...], preferred_element_type=_f32,
                   precision=lax.Precision.DEFAULT)

    @pl.when(k == 0)
    def _():
        y_ref[0] = part

    @pl.when(k > 0)
    def _():
        y_ref[0] = y_ref[0] + part

    @pl.when(k == nk - 1)
    def _():
        y_ref[0] = y_ref[0] * dinv_ref[...]


def _mm_mid(s_in, W, b_prev, dinv):
    nk = s_in.shape[0]
    nj = W.shape[1] // 128
    return pl.pallas_call(
        functools.partial(_mm_mid_body, nk=nk),
        grid=(NI, nj, nk),
        in_specs=[
            pl.BlockSpec((1, BM, 128), lambda i, j, k: (k, i, 0)),
            pl.BlockSpec((128, 128), lambda i, j, k: (k, j)),
            pl.BlockSpec((1, 128), lambda i, j, k: (0, k)),
            pl.BlockSpec((BM, 1), lambda i, j, k: (i, 0)),
        ],
        out_specs=pl.BlockSpec((1, BM, 128), lambda i, j, k: (j, i, 0)),
        out_shape=jax.ShapeDtypeStruct((nj, NPAD, 128), _f32),
    )(s_in, W, b_prev.reshape(1, -1), dinv)


def _pool_body(s_ref, dinv_ref, b3_ref, bidx_ref, out_ref, acc_ref, cnt_ref):
    i = pl.program_id(0)

    @pl.when(i == 0)
    def _():
        acc_ref[...] = jnp.zeros_like(acc_ref)
        cnt_ref[...] = jnp.zeros_like(cnt_ref)

    oh = (bidx_ref[...] == lax.broadcasted_iota(_i32, (BM, G), 1)).astype(_f32)
    dn = (((0,), (0,)), ((), ()))
    v0 = s_ref[0] * dinv_ref[...]
    v1 = s_ref[1] * dinv_ref[...]
    acc_ref[:, :128] = acc_ref[:, :128] + lax.dot_general(
        oh, v0, dn, preferred_element_type=_f32, precision=lax.Precision.DEFAULT)
    acc_ref[:, 128:] = acc_ref[:, 128:] + lax.dot_general(
        oh, v1, dn, preferred_element_type=_f32, precision=lax.Precision.DEFAULT)
    cnt_ref[...] = cnt_ref[...] + lax.dot_general(
        oh, jnp.ones((BM, 128), _f32), dn, preferred_element_type=_f32,
        precision=lax.Precision.DEFAULT)

    @pl.when(i == NI - 1)
    def _():
        cnt = cnt_ref[:, :1]
        out_ref[...] = (acc_ref[...] + cnt * b3_ref[...]) / jnp.maximum(cnt, 1.0)


def _pool(s3, dinv, b3, batch_p):
    return pl.pallas_call(
        _pool_body,
        grid=(NI,),
        in_specs=[
            pl.BlockSpec((2, BM, 128), lambda i: (0, i, 0)),
            pl.BlockSpec((BM, 1), lambda i: (i, 0)),
            pl.BlockSpec((1, F_OUT), lambda i: (0, 0)),
            pl.BlockSpec((BM, 1), lambda i: (i, 0)),
        ],
        out_specs=pl.BlockSpec((G, F_OUT), lambda i: (0, 0)),
        out_shape=jax.ShapeDtypeStruct((G, F_OUT), _f32),
        scratch_shapes=[
            pltpu.VMEM((G, F_OUT), _f32),
            pltpu.VMEM((G, 128), _f32),
        ],
    )(s3, dinv, b3.reshape(1, F_OUT), batch_p)


def kernel(x, edge_index, batch_index, W1, b1, W2, b2, W3, b3):
    src = edge_index[0]
    dst = edge_index[1]
    pad = jnp.full((EPAD - E,), NPAD - 1, _i32)
    src_p = jnp.concatenate([src, pad])
    dst_p = jnp.concatenate([dst, pad])
    src_r = src_p.reshape(NS, NHALF, EPH)
    dst_r = dst_p.reshape(NS, NHALF, NCHH, CW)

    x_p = jnp.pad(x, ((0, NPAD - N), (0, 0)))
    batch_p = jnp.pad(batch_index, (0, NPAD - N), constant_values=G)
    batch_p = batch_p.reshape(NPAD, 1)

    deg2 = _deg_call(dst_p)
    y1, dinv = _mm_first(x_p, W1, deg2)
    s1 = _prop_call(y1.reshape(-1, 128), src_r, dst_r, nb=4)
    y2 = _mm_mid(s1.reshape(4, NPAD, 128), W2, b1, dinv)
    s2 = _prop_call(y2.reshape(-1, 128), src_r, dst_r, nb=4)
    y3 = _mm_mid(s2.reshape(4, NPAD, 128), W3, b2, dinv)
    s3 = _prop_call(y3.reshape(-1, 128), src_r, dst_r, nb=2)
    return _pool(s3.reshape(2, NPAD, 128), dinv, b3, batch_p)
